# baseline (speedup 1.0000x reference)
import functools

import jax
import jax.numpy as jnp
from jax import lax
from jax.experimental import pallas as pl
from jax.experimental.pallas import tpu as pltpu
from jax.experimental.pallas import tpu_sc as plsc

NUM_LANES = 16
PK = 32
D = 128
DC = D // NUM_LANES
DP = D // PK
L_MAX = 200
L_SPLIT = 112
L_TAIL = L_MAX - L_SPLIT
L_PAD = 208
GLOBAL_MEAN = 3.5
INV_SQRT_D = 0.08838834764831845


def _lane_iota():
  return lax.iota(jnp.int32, NUM_LANES)


def _take16(v, idx):
  return lax.gather(
      v, idx[:, None],
      dimension_numbers=lax.GatherDimensionNumbers(
          offset_dims=(), collapsed_slice_dims=(0,), start_index_map=(0,)),
      slice_sizes=(1,),
      mode=lax.GatherScatterMode.PROMISE_IN_BOUNDS)


def _butterfly_sum(v):
  for s in (8, 4, 2, 1):
    v = v + _take16(v, _lane_iota() ^ s)
  return v


def _butterfly_max(v):
  for s in (8, 4, 2, 1):
    v = jnp.maximum(v, _take16(v, _lane_iota() ^ s))
  return v


def _splat(v, i):
  return _take16(v, jnp.full((NUM_LANES,), i, jnp.int32))


def _pack_row(ref, e, k):
  lo = ref[e, pl.ds(k * NUM_LANES, NUM_LANES)]
  hi = ref[e, pl.ds(k * NUM_LANES + D // 2, NUM_LANES)]
  return plsc.pack(lo, hi, format=plsc.PackFormat.INTERLEAVED)


def _unpack_sum(v_bf):
  a, b = plsc.unpack(v_bf, format=plsc.PackFormat.INTERLEAVED)
  return a + b


A_CH = 48


def _pack_phase(Y_hbm, ypk_hbm, nc, ns, st_v, ob_v, sem_in, sem_out):
  n_rows = Y_hbm.shape[0]
  per_sub = n_rows // 16
  a_lo = (per_sub * ns) & ~7
  a_hi = jnp.where(ns == 15, n_rows, (per_sub * (ns + 1)) & ~7)
  n_ch = (per_sub + 8 + A_CH - 1) // A_CH

  sems_in = sem_in
  sems_out = sem_out

  def r0_of(j):
    return jnp.minimum(a_lo + j * A_CH, a_hi - A_CH)

  def cp_in(j, b):
    return pltpu.make_async_copy(
        Y_hbm.at[pl.ds(r0_of(j), A_CH)], st_v.at[b], sems_in[b])

  def cp_out(j, b):
    return pltpu.make_async_copy(
        ob_v.at[b], ypk_hbm.at[nc, pl.ds(r0_of(j), A_CH)], sems_out[b])

  cp_in(0, 0).start()
  cp_in(1, 1).start()

  mask_hi = jnp.full((NUM_LANES,), 0xFFFF0000, jnp.uint32)

  def body(jj, _):
    for b in range(2):
      j = 2 * jj + b

      @pl.when(j < n_ch)
      def _():
        cp_in(j, b).wait()

        @pl.when(j >= 2)
        def _():
          cp_out(j - 2, b).wait()

        for r in range(A_CH):
          for k in range(DP):
            lo = plsc.bitcast(st_v[b, r, pl.ds(k * NUM_LANES, NUM_LANES)],
                              jnp.uint32)
            hi = plsc.bitcast(
                st_v[b, r, pl.ds(k * NUM_LANES + D // 2, NUM_LANES)],
                jnp.uint32)
            ob_v[b, r, pl.ds(k * NUM_LANES, NUM_LANES)] = plsc.bitcast(
                (lo >> 16) | (hi & mask_hi), jnp.int32)

        cp_out(j, b).start()

        @pl.when(j + 2 < n_ch)
        def _():
          cp_in(j + 2, b).start()
    return 0

  lax.fori_loop(0, (n_ch + 1) // 2, body, 0, unroll=False)
  cp_out(n_ch - 2, (n_ch - 2) % 2).wait()
  cp_out(n_ch - 1, (n_ch - 1) % 2).wait()
  plsc.subcore_barrier()


def _body(n_elems, P_hbm, Q_hbm, Y_hbm, Bs_hbm, Bp_hbm, gw_hbm, gb_hbm,
          sid_hbm, pid_hbm, ipid_hbm, len_hbm, out_hbm, ypk_hbm,
          sid_v, pid_v, len_v, ids_v, p_v, q_v, bs_v, bp_v, gw_v, gb_v,
          y_v, scores_v, out_v, st_v, ob_v,
          sem_pq, sem_y0, sem_y1, sem_ain0, sem_ain1,
          sem_aout0, sem_aout1):
  nc = lax.axis_index("c")
  ns = lax.axis_index("s")
  wid = ns * 2 + nc
  base = wid * n_elems

  _pack_phase(Y_hbm, ypk_hbm, nc, ns, st_v, ob_v,
              (sem_ain0, sem_ain1), (sem_aout0, sem_aout1))

  pltpu.sync_copy(sid_hbm.at[pl.ds(base, n_elems)], sid_v)
  pltpu.sync_copy(pid_hbm.at[pl.ds(base, n_elems)], pid_v)
  pltpu.sync_copy(ipid_hbm.at[pl.ds(base, n_elems), pl.ds(0, L_SPLIT)],
                  ids_v.at[:, 0])
  pltpu.sync_copy(ipid_hbm.at[pl.ds(base, n_elems), pl.ds(L_SPLIT, L_TAIL)],
                  ids_v.at[:, 1, pl.ds(0, L_TAIL)])
  pltpu.sync_copy(gw_hbm, gw_v)
  pltpu.sync_copy(gb_hbm, gb_v)
  pltpu.sync_copy(len_hbm.at[pl.ds(base, n_elems)],
                  len_v.at[pl.ds(0, n_elems)])
  cp_p = pltpu.make_async_copy(P_hbm.at[sid_v], p_v, sem_pq)
  cp_q = pltpu.make_async_copy(Q_hbm.at[pid_v], q_v, sem_pq)
  cp_bs = pltpu.make_async_copy(Bs_hbm.at[sid_v], bs_v, sem_pq)
  cp_bp = pltpu.make_async_copy(Bp_hbm.at[pid_v], bp_v, sem_pq)
  cp_p.start(); cp_q.start(); cp_bs.start(); cp_bp.start()

  zrow = jnp.zeros((NUM_LANES,), jnp.int32)
  for b in range(2):
    for r in range(L_MAX, L_PAD):
      for k in range(DP):
        y_v[b, r, pl.ds(k * NUM_LANES, NUM_LANES)] = zrow

  cp_p.wait(); cp_q.wait(); cp_bs.wait(); cp_bp.wait()

  sems = (sem_y0, sem_y1)
  y_words = ypk_hbm.at[nc]

  def _half_copy(e, b, h):
    if h == 0:
      return pltpu.make_async_copy(
          y_words.at[ids_v.at[e, 0]],
          y_v.at[b, pl.ds(0, L_SPLIT)],
          sems[b])
    return pltpu.make_async_copy(
        y_words.at[ids_v.at[e, 1, pl.ds(0, L_TAIL)]],
        y_v.at[b, pl.ds(L_SPLIT, L_TAIL)],
        sems[b])

  def _elem_len(e):
    return len_v[pl.ds(e, NUM_LANES)][0]

  def start_gather(e, b):
    _half_copy(e, b, 0).start()

    @pl.when(_elem_len(e) > L_SPLIT)
    def _():
      _half_copy(e, b, 1).start()

  def wait_gather(e, b):
    _half_copy(e, b, 0).wait()

    @pl.when(_elem_len(e) > L_SPLIT)
    def _():
      _half_copy(e, b, 1).wait()

  start_gather(0, 0)
  start_gather(1, 1)

  lane = _lane_iota()

  def compute(e, b, out_chunk):
    ln = len_v[pl.ds(e, NUM_LANES)][0]
    nch = (ln + (NUM_LANES - 1)) // NUM_LANES

    p_pk = [_pack_row(p_v, e, k) for k in range(DP)]

    def y_chunk(l, k):
      return plsc.bitcast(y_v[b, l, pl.ds(k * NUM_LANES, NUM_LANES)],
                          jnp.bfloat16)

    def score_chunk_body(c, m):
      l0 = c * NUM_LANES
      chunk = jnp.zeros((NUM_LANES,), jnp.float32)
      for i in range(NUM_LANES):
        acc = y_chunk(l0 + i, 0) * p_pk[0]
        for k in range(1, DP):
          acc = acc + y_chunk(l0 + i, k) * p_pk[k]
        s = jnp.sum(_unpack_sum(acc))
        chunk = jnp.where(lane == i, s, chunk)
      chunk = chunk * INV_SQRT_D
      scores_v[pl.ds(l0, NUM_LANES)] = chunk
      valid = (l0 + lane) < ln
      return jnp.maximum(m, jnp.where(valid, chunk, -jnp.inf))
    m_vec = lax.fori_loop(0, nch, score_chunk_body,
                          jnp.full((NUM_LANES,), -jnp.inf, jnp.float32),
                          unroll=False)
    m = jnp.max(m_vec)

    def wsum_body(c, carry):
      *accs, dn = carry
      l0 = c * NUM_LANES
      sraw = scores_v[pl.ds(l0, NUM_LANES)]
      valid = (l0 + lane) < ln
      w_chunk = jnp.where(valid, jnp.exp(sraw - m), 0.0)
      dn = dn + w_chunk
      accs = list(accs)
      for i in range(NUM_LANES):
        w_i = _splat(w_chunk, i)
        w_pk = plsc.pack(w_i, w_i, format=plsc.PackFormat.INTERLEAVED)
        for k in range(DP):
          accs[k] = accs[k] + w_pk * y_chunk(l0 + i, k)
      return (*accs, dn)
    *accs, dn = lax.fori_loop(
        0, nch, wsum_body,
        (*(jnp.zeros((PK,), jnp.bfloat16) for _ in range(DP)),
         jnp.zeros((NUM_LANES,), jnp.float32)),
        unroll=False)
    denom = jnp.sum(dn)
    inv_dn = 1.0 / jnp.broadcast_to(denom, (NUM_LANES,))
    dn_pk = plsc.pack(inv_dn, inv_dn, format=plsc.PackFormat.INTERLEAVED)
    s_pk = [a * dn_pk for a in accs]

    g_acc = p_pk[0] * _pack_row(gw_v, 0, 0)
    for k in range(1, DP):
      g_acc = g_acc + p_pk[k] * _pack_row(gw_v, 0, k)
    for k in range(DP):
      g_acc = g_acc + s_pk[k] * _pack_row(gw_v, 1, k)
    x = jnp.broadcast_to(jnp.sum(_unpack_sum(g_acc)), (NUM_LANES,)) + gb_v[...]
    g = 1.0 / (1.0 + jnp.exp(-x))
    g_pk = plsc.pack(g, g, format=plsc.PackFormat.INTERLEAVED)

    t = jnp.zeros((PK,), jnp.bfloat16)
    for k in range(DP):
      u_k = s_pk[k] + g_pk * (p_pk[k] - s_pk[k])
      t = t + _pack_row(q_v, e, k) * u_k
    inter = jnp.sum(_unpack_sum(t))

    return jnp.where(lane == (e % NUM_LANES), inter, out_chunk)

  def outer(j, out_chunk):
    for b in range(2):
      e = 2 * j + b
      wait_gather(e, b)
      out_chunk = compute(e, b, out_chunk)

      @pl.when(e + 2 < n_elems)
      def _():
        start_gather(e + 2, b)

      @pl.when(e % NUM_LANES == NUM_LANES - 1)
      def _():
        g0 = (e // NUM_LANES) * NUM_LANES
        out_v[pl.ds(g0, NUM_LANES)] = (
            out_chunk + bs_v[pl.ds(g0, NUM_LANES)]
            + bp_v[pl.ds(g0, NUM_LANES)] + GLOBAL_MEAN)
    return out_chunk

  lax.fori_loop(0, n_elems // 2, outer,
                jnp.zeros((NUM_LANES,), jnp.float32), unroll=False)

  pltpu.sync_copy(out_v, out_hbm.at[pl.ds(base, n_elems)])


def kernel(P, Q, Y, Bs, Bp, gate_W, gate_b, SIDs, PIDs, implicit_PIDs,
           implicit_lengths):
  B = SIDs.shape[0]
  n_workers = 32
  n_elems = B // n_workers

  sids = SIDs.astype(jnp.int32)
  pids = PIDs.astype(jnp.int32)
  lens = implicit_lengths.astype(jnp.int32)
  ipids = implicit_PIDs.astype(jnp.int32)
  gw = gate_W.reshape(2, D)
  gb = jnp.broadcast_to(gate_b.reshape(1), (NUM_LANES,)).astype(jnp.float32)

  mesh = plsc.VectorSubcoreMesh(core_axis_name="c", subcore_axis_name="s")
  f = pl.kernel(
      functools.partial(_body, n_elems),
      out_type=(jax.ShapeDtypeStruct((B,), jnp.float32),
                jax.ShapeDtypeStruct((2, Y.shape[0], D // 2), jnp.int32)),
      mesh=mesh,
      compiler_params=pltpu.CompilerParams(needs_layout_passes=False,
                                           use_tc_tiling_on_sc=False),
      scratch_types=[
          pltpu.VMEM((n_elems,), jnp.int32),
          pltpu.VMEM((n_elems,), jnp.int32),
          pltpu.VMEM((n_elems + NUM_LANES,), jnp.int32),
          pltpu.VMEM((n_elems, 2, L_SPLIT), jnp.int32),
          pltpu.VMEM((n_elems, D), jnp.float32),
          pltpu.VMEM((n_elems, D), jnp.float32),
          pltpu.VMEM((n_elems,), jnp.float32),
          pltpu.VMEM((n_elems,), jnp.float32),
          pltpu.VMEM((2, D), jnp.float32),
          pltpu.VMEM((NUM_LANES,), jnp.float32),
          pltpu.VMEM((2, L_PAD, D // 2), jnp.int32),
          pltpu.VMEM((L_PAD,), jnp.float32),
          pltpu.VMEM((n_elems,), jnp.float32),
          pltpu.VMEM((2, A_CH, D), jnp.float32),
          pltpu.VMEM((2, A_CH, D // 2), jnp.int32),
          pltpu.SemaphoreType.DMA,
          pltpu.SemaphoreType.DMA,
          pltpu.SemaphoreType.DMA,
          pltpu.SemaphoreType.DMA,
          pltpu.SemaphoreType.DMA,
          pltpu.SemaphoreType.DMA,
          pltpu.SemaphoreType.DMA,
      ],
  )
  pred, _ = f(P, Q, Y, Bs.reshape(-1), Bp.reshape(-1), gw, gb, sids, pids,
              ipids, lens)
  return pred


# speedup vs baseline: 1.0175x; 1.0175x over previous
import functools

import jax
import jax.numpy as jnp
from jax import lax
from jax.experimental import pallas as pl
from jax.experimental.pallas import tpu as pltpu
from jax.experimental.pallas import tpu_sc as plsc

NUM_LANES = 16
PK = 32
D = 128
DC = D // NUM_LANES
DP = D // PK
L_MAX = 200
L_SPLIT = 112
L_TAIL = L_MAX - L_SPLIT
L_PAD = 208
GLOBAL_MEAN = 3.5
INV_SQRT_D = 0.08838834764831845


def _lane_iota():
  return lax.iota(jnp.int32, NUM_LANES)


def _take16(v, idx):
  return lax.gather(
      v, idx[:, None],
      dimension_numbers=lax.GatherDimensionNumbers(
          offset_dims=(), collapsed_slice_dims=(0,), start_index_map=(0,)),
      slice_sizes=(1,),
      mode=lax.GatherScatterMode.PROMISE_IN_BOUNDS)


def _butterfly_sum(v):
  for s in (8, 4, 2, 1):
    v = v + _take16(v, _lane_iota() ^ s)
  return v


def _butterfly_max(v):
  for s in (8, 4, 2, 1):
    v = jnp.maximum(v, _take16(v, _lane_iota() ^ s))
  return v


def _splat(v, i):
  return _take16(v, jnp.full((NUM_LANES,), i, jnp.int32))


def _pack_row(ref, e, k):
  lo = ref[e, pl.ds(k * NUM_LANES, NUM_LANES)]
  hi = ref[e, pl.ds(k * NUM_LANES + D // 2, NUM_LANES)]
  return plsc.pack(lo, hi, format=plsc.PackFormat.INTERLEAVED)


def _unpack_sum(v_bf):
  a, b = plsc.unpack(v_bf, format=plsc.PackFormat.INTERLEAVED)
  return a + b


A_CH = 48


def _pack_phase(Y_hbm, ypk_hbm, nc, ns, st_v, ob_v, sem_in, sem_out):
  n_rows = Y_hbm.shape[0]
  per_sub = n_rows // 16
  a_lo = (per_sub * ns) & ~7
  a_hi = jnp.where(ns == 15, n_rows, (per_sub * (ns + 1)) & ~7)
  n_ch = (per_sub + 8 + A_CH - 1) // A_CH

  sems_in = sem_in
  sems_out = sem_out

  def r0_of(j):
    return jnp.minimum(a_lo + j * A_CH, a_hi - A_CH)

  def cp_in(j, b):
    return pltpu.make_async_copy(
        Y_hbm.at[pl.ds(r0_of(j), A_CH)], st_v.at[b], sems_in[b])

  def cp_out(j, b):
    return pltpu.make_async_copy(
        ob_v.at[b], ypk_hbm.at[nc, pl.ds(r0_of(j), A_CH)], sems_out[b])

  cp_in(0, 0).start()
  cp_in(1, 1).start()

  mask_hi = jnp.full((NUM_LANES,), 0xFFFF0000, jnp.uint32)

  def body(jj, _):
    for b in range(2):
      j = 2 * jj + b

      @pl.when(j < n_ch)
      def _():
        cp_in(j, b).wait()

        @pl.when(j >= 2)
        def _():
          cp_out(j - 2, b).wait()

        for r in range(A_CH):
          for k in range(DP):
            lo = plsc.bitcast(st_v[b, r, pl.ds(k * NUM_LANES, NUM_LANES)],
                              jnp.uint32)
            hi = plsc.bitcast(
                st_v[b, r, pl.ds(k * NUM_LANES + D // 2, NUM_LANES)],
                jnp.uint32)
            ob_v[b, r, pl.ds(k * NUM_LANES, NUM_LANES)] = plsc.bitcast(
                (lo >> 16) | (hi & mask_hi), jnp.int32)

        cp_out(j, b).start()

        @pl.when(j + 2 < n_ch)
        def _():
          cp_in(j + 2, b).start()
    return 0

  lax.fori_loop(0, (n_ch + 1) // 2, body, 0, unroll=False)
  cp_out(n_ch - 2, (n_ch - 2) % 2).wait()
  cp_out(n_ch - 1, (n_ch - 1) % 2).wait()
  plsc.subcore_barrier()


def _body(n_elems, P_hbm, Q_hbm, Y_hbm, Bs_hbm, Bp_hbm, gw_hbm, gb_hbm,
          sid_hbm, pid_hbm, ipid_hbm, len_hbm, out_hbm, ypk_hbm,
          sid_v, pid_v, len_v, ids_v, p_v, q_v, bs_v, bp_v, gw_v, gb_v,
          y_v, scores_v, out_v, st_v, ob_v,
          sem_pq, sem_y0, sem_y1, sem_ain0, sem_ain1,
          sem_aout0, sem_aout1):
  nc = lax.axis_index("c")
  ns = lax.axis_index("s")
  wid = ns * 2 + nc
  base = wid * n_elems

  _pack_phase(Y_hbm, ypk_hbm, nc, ns, st_v, ob_v,
              (sem_ain0, sem_ain1), (sem_aout0, sem_aout1))

  pltpu.sync_copy(sid_hbm.at[pl.ds(base, n_elems)], sid_v)
  pltpu.sync_copy(pid_hbm.at[pl.ds(base, n_elems)], pid_v)
  pltpu.sync_copy(ipid_hbm.at[pl.ds(base, n_elems), pl.ds(0, L_SPLIT)],
                  ids_v.at[:, 0])
  pltpu.sync_copy(ipid_hbm.at[pl.ds(base, n_elems), pl.ds(L_SPLIT, L_TAIL)],
                  ids_v.at[:, 1, pl.ds(0, L_TAIL)])
  pltpu.sync_copy(gw_hbm, gw_v)
  pltpu.sync_copy(gb_hbm, gb_v)
  pltpu.sync_copy(len_hbm.at[pl.ds(base, n_elems)],
                  len_v.at[pl.ds(0, n_elems)])
  cp_p = pltpu.make_async_copy(P_hbm.at[sid_v], p_v, sem_pq)
  cp_q = pltpu.make_async_copy(Q_hbm.at[pid_v], q_v, sem_pq)
  cp_bs = pltpu.make_async_copy(Bs_hbm.at[sid_v], bs_v, sem_pq)
  cp_bp = pltpu.make_async_copy(Bp_hbm.at[pid_v], bp_v, sem_pq)
  cp_p.start(); cp_q.start(); cp_bs.start(); cp_bp.start()

  zrow = jnp.zeros((NUM_LANES,), jnp.int32)
  for b in range(2):
    for r in range(L_MAX, L_PAD):
      for k in range(DP):
        y_v[b, r, pl.ds(k * NUM_LANES, NUM_LANES)] = zrow

  cp_p.wait(); cp_q.wait(); cp_bs.wait(); cp_bp.wait()

  sems = (sem_y0, sem_y1)
  y_words = ypk_hbm.at[nc]

  def _half_copy(e, b, h):
    if h == 0:
      return pltpu.make_async_copy(
          y_words.at[ids_v.at[e, 0]],
          y_v.at[b, pl.ds(0, L_SPLIT)],
          sems[b])
    return pltpu.make_async_copy(
        y_words.at[ids_v.at[e, 1, pl.ds(0, L_TAIL)]],
        y_v.at[b, pl.ds(L_SPLIT, L_TAIL)],
        sems[b])

  def _elem_len(e):
    return len_v[pl.ds(e, NUM_LANES)][0]

  def start_gather(e, b):
    _half_copy(e, b, 0).start()

    @pl.when(_elem_len(e) > L_SPLIT)
    def _():
      _half_copy(e, b, 1).start()

  def wait_gather(e, b):
    _half_copy(e, b, 0).wait()

    @pl.when(_elem_len(e) > L_SPLIT)
    def _():
      _half_copy(e, b, 1).wait()

  start_gather(0, 0)
  start_gather(1, 1)

  lane = _lane_iota()

  def compute(e, b, out_chunk):
    ln = len_v[pl.ds(e, NUM_LANES)][0]
    nch = (ln + (NUM_LANES - 1)) // NUM_LANES

    p_pk = [_pack_row(p_v, e, k) for k in range(DP)]

    def y_chunk(l, k):
      return plsc.bitcast(y_v[b, l, pl.ds(k * NUM_LANES, NUM_LANES)],
                          jnp.bfloat16)

    def score_chunk_body(c, _):
      l0 = c * NUM_LANES
      chunk = jnp.zeros((NUM_LANES,), jnp.float32)
      for i in range(NUM_LANES):
        acc = y_chunk(l0 + i, 0) * p_pk[0]
        for k in range(1, DP):
          acc = acc + y_chunk(l0 + i, k) * p_pk[k]
        s = jnp.sum(_unpack_sum(acc))
        chunk = jnp.where(lane == i, s, chunk)
      scores_v[pl.ds(l0, NUM_LANES)] = chunk * INV_SQRT_D
      return 0
    lax.fori_loop(0, nch, score_chunk_body, 0, unroll=False)

    def max_body(c, m):
      s = scores_v[pl.ds(c * NUM_LANES, NUM_LANES)]
      valid = (c * NUM_LANES + lane) < ln
      return jnp.maximum(m, jnp.where(valid, s, -jnp.inf))
    m_vec = lax.fori_loop(0, nch, max_body,
                          jnp.full((NUM_LANES,), -jnp.inf, jnp.float32))
    m = jnp.max(m_vec)

    def exp_body(c, dn):
      s = scores_v[pl.ds(c * NUM_LANES, NUM_LANES)]
      valid = (c * NUM_LANES + lane) < ln
      w = jnp.where(valid, jnp.exp(s - m), 0.0)
      scores_v[pl.ds(c * NUM_LANES, NUM_LANES)] = w
      return dn + w
    denom = lax.fori_loop(0, nch, exp_body,
                          jnp.zeros((NUM_LANES,), jnp.float32))
    denom = jnp.sum(denom)

    def wsum_body(c, accs):
      l0 = c * NUM_LANES
      w_chunk = scores_v[pl.ds(l0, NUM_LANES)]
      accs = list(accs)
      for i in range(NUM_LANES):
        w_i = _splat(w_chunk, i)
        w_pk = plsc.pack(w_i, w_i, format=plsc.PackFormat.INTERLEAVED)
        for k in range(DP):
          accs[k] = accs[k] + w_pk * y_chunk(l0 + i, k)
      return tuple(accs)
    accs = lax.fori_loop(
        0, nch, wsum_body,
        tuple(jnp.zeros((PK,), jnp.bfloat16) for _ in range(DP)),
        unroll=False)
    inv_dn = 1.0 / jnp.broadcast_to(denom, (NUM_LANES,))
    dn_pk = plsc.pack(inv_dn, inv_dn, format=plsc.PackFormat.INTERLEAVED)
    s_pk = [a * dn_pk for a in accs]

    g_acc = p_pk[0] * _pack_row(gw_v, 0, 0)
    for k in range(1, DP):
      g_acc = g_acc + p_pk[k] * _pack_row(gw_v, 0, k)
    for k in range(DP):
      g_acc = g_acc + s_pk[k] * _pack_row(gw_v, 1, k)
    x = jnp.broadcast_to(jnp.sum(_unpack_sum(g_acc)), (NUM_LANES,)) + gb_v[...]
    g = 1.0 / (1.0 + jnp.exp(-x))
    g_pk = plsc.pack(g, g, format=plsc.PackFormat.INTERLEAVED)

    t = jnp.zeros((PK,), jnp.bfloat16)
    for k in range(DP):
      u_k = s_pk[k] + g_pk * (p_pk[k] - s_pk[k])
      t = t + _pack_row(q_v, e, k) * u_k
    inter = jnp.sum(_unpack_sum(t))

    return jnp.where(lane == (e % NUM_LANES), inter, out_chunk)

  def outer(j, out_chunk):
    for b in range(2):
      e = 2 * j + b
      wait_gather(e, b)
      out_chunk = compute(e, b, out_chunk)

      @pl.when(e + 2 < n_elems)
      def _():
        start_gather(e + 2, b)

      @pl.when(e % NUM_LANES == NUM_LANES - 1)
      def _():
        g0 = (e // NUM_LANES) * NUM_LANES
        out_v[pl.ds(g0, NUM_LANES)] = (
            out_chunk + bs_v[pl.ds(g0, NUM_LANES)]
            + bp_v[pl.ds(g0, NUM_LANES)] + GLOBAL_MEAN)
    return out_chunk

  lax.fori_loop(0, n_elems // 2, outer,
                jnp.zeros((NUM_LANES,), jnp.float32), unroll=False)

  pltpu.sync_copy(out_v, out_hbm.at[pl.ds(base, n_elems)])


def kernel(P, Q, Y, Bs, Bp, gate_W, gate_b, SIDs, PIDs, implicit_PIDs,
           implicit_lengths):
  B = SIDs.shape[0]
  n_workers = 32
  n_elems = B // n_workers

  sids = SIDs.astype(jnp.int32)
  pids = PIDs.astype(jnp.int32)
  lens = implicit_lengths.astype(jnp.int32)
  ipids = implicit_PIDs.astype(jnp.int32)
  gw = gate_W.reshape(2, D)
  gb = jnp.broadcast_to(gate_b.reshape(1), (NUM_LANES,)).astype(jnp.float32)

  mesh = plsc.VectorSubcoreMesh(core_axis_name="c", subcore_axis_name="s")
  f = pl.kernel(
      functools.partial(_body, n_elems),
      out_type=(jax.ShapeDtypeStruct((B,), jnp.float32),
                jax.ShapeDtypeStruct((2, Y.shape[0], D // 2), jnp.int32)),
      mesh=mesh,
      compiler_params=pltpu.CompilerParams(needs_layout_passes=False,
                                           use_tc_tiling_on_sc=False),
      scratch_types=[
          pltpu.VMEM((n_elems,), jnp.int32),
          pltpu.VMEM((n_elems,), jnp.int32),
          pltpu.VMEM((n_elems + NUM_LANES,), jnp.int32),
          pltpu.VMEM((n_elems, 2, L_SPLIT), jnp.int32),
          pltpu.VMEM((n_elems, D), jnp.float32),
          pltpu.VMEM((n_elems, D), jnp.float32),
          pltpu.VMEM((n_elems,), jnp.float32),
          pltpu.VMEM((n_elems,), jnp.float32),
          pltpu.VMEM((2, D), jnp.float32),
          pltpu.VMEM((NUM_LANES,), jnp.float32),
          pltpu.VMEM((2, L_PAD, D // 2), jnp.int32),
          pltpu.VMEM((L_PAD,), jnp.float32),
          pltpu.VMEM((n_elems,), jnp.float32),
          pltpu.VMEM((2, A_CH, D), jnp.float32),
          pltpu.VMEM((2, A_CH, D // 2), jnp.int32),
          pltpu.SemaphoreType.DMA,
          pltpu.SemaphoreType.DMA,
          pltpu.SemaphoreType.DMA,
          pltpu.SemaphoreType.DMA,
          pltpu.SemaphoreType.DMA,
          pltpu.SemaphoreType.DMA,
          pltpu.SemaphoreType.DMA,
      ],
  )
  pred, _ = f(P, Q, Y, Bs.reshape(-1), Bp.reshape(-1), gw, gb, sids, pids,
              ipids, lens)
  return pred


# speedup vs baseline: 1.2377x; 1.2164x over previous
import functools

import jax
import jax.numpy as jnp
from jax import lax
from jax.experimental import pallas as pl
from jax.experimental.pallas import tpu as pltpu
from jax.experimental.pallas import tpu_sc as plsc

NUM_LANES = 16
PK = 32
D = 128
DC = D // NUM_LANES
DP = D // PK
L_MAX = 200
L_SPLIT = 112
L_TAIL = L_MAX - L_SPLIT
L_PAD = 208
GLOBAL_MEAN = 3.5
INV_SQRT_D = 0.08838834764831845


def _lane_iota():
  return lax.iota(jnp.int32, NUM_LANES)


def _take16(v, idx):
  return lax.gather(
      v, idx[:, None],
      dimension_numbers=lax.GatherDimensionNumbers(
          offset_dims=(), collapsed_slice_dims=(0,), start_index_map=(0,)),
      slice_sizes=(1,),
      mode=lax.GatherScatterMode.PROMISE_IN_BOUNDS)


def _butterfly_sum(v):
  for s in (8, 4, 2, 1):
    v = v + _take16(v, _lane_iota() ^ s)
  return v


def _butterfly_max(v):
  for s in (8, 4, 2, 1):
    v = jnp.maximum(v, _take16(v, _lane_iota() ^ s))
  return v


def _splat(v, i):
  return _take16(v, jnp.full((NUM_LANES,), i, jnp.int32))


def _pack_row(ref, e, k):
  lo = ref[e, pl.ds(k * NUM_LANES, NUM_LANES)]
  hi = ref[e, pl.ds(k * NUM_LANES + D // 2, NUM_LANES)]
  return plsc.pack(lo, hi, format=plsc.PackFormat.INTERLEAVED)


def _unpack_sum(v_bf):
  a, b = plsc.unpack(v_bf, format=plsc.PackFormat.INTERLEAVED)
  return a + b


A_CH = 48


def _pack_body(Y_hbm, ypk_hbm, st_v, ob_v, sem_in0, sem_in1,
               sem_out0, sem_out1):
  nc = lax.axis_index("c")
  ns = lax.axis_index("s")
  wid = ns * 2 + nc
  n_rows = Y_hbm.shape[0]
  per_sub = n_rows // 32
  a_lo = (per_sub * wid) & ~7
  a_hi = jnp.where(wid == 31, n_rows, (per_sub * (wid + 1)) & ~7)
  n_ch = (per_sub + 8 + A_CH - 1) // A_CH

  sems_in = (sem_in0, sem_in1)
  sems_out = (sem_out0, sem_out1)

  def r0_of(j):
    return jnp.minimum(a_lo + j * A_CH, a_hi - A_CH)

  def cp_in(j, b):
    return pltpu.make_async_copy(
        Y_hbm.at[pl.ds(r0_of(j), A_CH)], st_v.at[b], sems_in[b])

  def cp_out(j, b):
    return pltpu.make_async_copy(
        ob_v.at[b], ypk_hbm.at[pl.ds(r0_of(j), A_CH)], sems_out[b])

  cp_in(0, 0).start()
  cp_in(1, 1).start()

  mask_hi = jnp.full((NUM_LANES,), 0xFFFF0000, jnp.uint32)

  def body(jj, _):
    for b in range(2):
      j = 2 * jj + b

      @pl.when(j < n_ch)
      def _():
        cp_in(j, b).wait()

        @pl.when(j >= 2)
        def _():
          cp_out(j - 2, b).wait()

        for r in range(A_CH):
          for k in range(DP):
            lo = plsc.bitcast(st_v[b, r, pl.ds(k * NUM_LANES, NUM_LANES)],
                              jnp.uint32)
            hi = plsc.bitcast(
                st_v[b, r, pl.ds(k * NUM_LANES + D // 2, NUM_LANES)],
                jnp.uint32)
            ob_v[b, r, pl.ds(k * NUM_LANES, NUM_LANES)] = plsc.bitcast(
                (lo >> 16) | (hi & mask_hi), jnp.int32)

        cp_out(j, b).start()

        @pl.when(j + 2 < n_ch)
        def _():
          cp_in(j + 2, b).start()
    return 0

  lax.fori_loop(0, (n_ch + 1) // 2, body, 0, unroll=False)
  cp_out(n_ch - 2, (n_ch - 2) % 2).wait()
  cp_out(n_ch - 1, (n_ch - 1) % 2).wait()


def _body(n_elems, P_hbm, Q_hbm, ypk_hbm, Bs_hbm, Bp_hbm, gw_hbm, gb_hbm,
          sid_hbm, pid_hbm, ipid_hbm, len_hbm, out_hbm,
          sid_v, pid_v, len_v, ids_v, p_v, q_v, bs_v, bp_v, gw_v, gb_v,
          y_v, scores_v, out_v, sem_pq, sem_y0, sem_y1):
  nc = lax.axis_index("c")
  ns = lax.axis_index("s")
  wid = ns * 2 + nc
  base = wid * n_elems

  pltpu.sync_copy(sid_hbm.at[pl.ds(base, n_elems)], sid_v)
  pltpu.sync_copy(pid_hbm.at[pl.ds(base, n_elems)], pid_v)
  pltpu.sync_copy(ipid_hbm.at[pl.ds(base, n_elems), pl.ds(0, L_SPLIT)],
                  ids_v.at[:, 0])
  pltpu.sync_copy(ipid_hbm.at[pl.ds(base, n_elems), pl.ds(L_SPLIT, L_TAIL)],
                  ids_v.at[:, 1, pl.ds(0, L_TAIL)])
  pltpu.sync_copy(gw_hbm, gw_v)
  pltpu.sync_copy(gb_hbm, gb_v)
  pltpu.sync_copy(len_hbm.at[pl.ds(base, n_elems)],
                  len_v.at[pl.ds(0, n_elems)])
  cp_p = pltpu.make_async_copy(P_hbm.at[sid_v], p_v, sem_pq)
  cp_q = pltpu.make_async_copy(Q_hbm.at[pid_v], q_v, sem_pq)
  cp_bs = pltpu.make_async_copy(Bs_hbm.at[sid_v], bs_v, sem_pq)
  cp_bp = pltpu.make_async_copy(Bp_hbm.at[pid_v], bp_v, sem_pq)
  cp_p.start(); cp_q.start(); cp_bs.start(); cp_bp.start()

  zrow = jnp.zeros((NUM_LANES,), jnp.int32)
  for b in range(2):
    for r in range(L_MAX, L_PAD):
      for k in range(DP):
        y_v[b, r, pl.ds(k * NUM_LANES, NUM_LANES)] = zrow

  cp_p.wait(); cp_q.wait(); cp_bs.wait(); cp_bp.wait()

  sems = (sem_y0, sem_y1)
  y_words = ypk_hbm

  def _half_copy(e, b, h):
    if h == 0:
      return pltpu.make_async_copy(
          y_words.at[ids_v.at[e, 0]],
          y_v.at[b, pl.ds(0, L_SPLIT)],
          sems[b])
    return pltpu.make_async_copy(
        y_words.at[ids_v.at[e, 1, pl.ds(0, L_TAIL)]],
        y_v.at[b, pl.ds(L_SPLIT, L_TAIL)],
        sems[b])

  def _elem_len(e):
    return len_v[pl.ds(e, NUM_LANES)][0]

  def start_gather(e, b):
    _half_copy(e, b, 0).start()

    @pl.when(_elem_len(e) > L_SPLIT)
    def _():
      _half_copy(e, b, 1).start()

  def wait_gather(e, b):
    _half_copy(e, b, 0).wait()

    @pl.when(_elem_len(e) > L_SPLIT)
    def _():
      _half_copy(e, b, 1).wait()

  start_gather(0, 0)
  start_gather(1, 1)

  lane = _lane_iota()

  def compute(e, b, out_chunk):
    ln = len_v[pl.ds(e, NUM_LANES)][0]
    nch = (ln + (NUM_LANES - 1)) // NUM_LANES

    p_pk = [_pack_row(p_v, e, k) for k in range(DP)]

    def y_chunk(l, k):
      return plsc.bitcast(y_v[b, l, pl.ds(k * NUM_LANES, NUM_LANES)],
                          jnp.bfloat16)

    def score_chunk_body(c, _):
      l0 = c * NUM_LANES
      chunk = jnp.zeros((NUM_LANES,), jnp.float32)
      for i in range(NUM_LANES):
        acc = y_chunk(l0 + i, 0) * p_pk[0]
        for k in range(1, DP):
          acc = acc + y_chunk(l0 + i, k) * p_pk[k]
        s = jnp.sum(_unpack_sum(acc))
        chunk = jnp.where(lane == i, s, chunk)
      scores_v[pl.ds(l0, NUM_LANES)] = chunk * INV_SQRT_D
      return 0
    lax.fori_loop(0, nch, score_chunk_body, 0, unroll=False)

    def max_body(c, m):
      s = scores_v[pl.ds(c * NUM_LANES, NUM_LANES)]
      valid = (c * NUM_LANES + lane) < ln
      return jnp.maximum(m, jnp.where(valid, s, -jnp.inf))
    m_vec = lax.fori_loop(0, nch, max_body,
                          jnp.full((NUM_LANES,), -jnp.inf, jnp.float32))
    m = jnp.max(m_vec)

    def exp_body(c, dn):
      s = scores_v[pl.ds(c * NUM_LANES, NUM_LANES)]
      valid = (c * NUM_LANES + lane) < ln
      w = jnp.where(valid, jnp.exp(s - m), 0.0)
      scores_v[pl.ds(c * NUM_LANES, NUM_LANES)] = w
      return dn + w
    denom = lax.fori_loop(0, nch, exp_body,
                          jnp.zeros((NUM_LANES,), jnp.float32))
    denom = jnp.sum(denom)

    def wsum_body(c, accs):
      l0 = c * NUM_LANES
      w_chunk = scores_v[pl.ds(l0, NUM_LANES)]
      accs = list(accs)
      for i in range(NUM_LANES):
        w_i = _splat(w_chunk, i)
        w_pk = plsc.pack(w_i, w_i, format=plsc.PackFormat.INTERLEAVED)
        for k in range(DP):
          accs[k] = accs[k] + w_pk * y_chunk(l0 + i, k)
      return tuple(accs)
    accs = lax.fori_loop(
        0, nch, wsum_body,
        tuple(jnp.zeros((PK,), jnp.bfloat16) for _ in range(DP)),
        unroll=False)
    inv_dn = 1.0 / jnp.broadcast_to(denom, (NUM_LANES,))
    dn_pk = plsc.pack(inv_dn, inv_dn, format=plsc.PackFormat.INTERLEAVED)
    s_pk = [a * dn_pk for a in accs]

    g_acc = p_pk[0] * _pack_row(gw_v, 0, 0)
    for k in range(1, DP):
      g_acc = g_acc + p_pk[k] * _pack_row(gw_v, 0, k)
    for k in range(DP):
      g_acc = g_acc + s_pk[k] * _pack_row(gw_v, 1, k)
    x = jnp.broadcast_to(jnp.sum(_unpack_sum(g_acc)), (NUM_LANES,)) + gb_v[...]
    g = 1.0 / (1.0 + jnp.exp(-x))
    g_pk = plsc.pack(g, g, format=plsc.PackFormat.INTERLEAVED)

    t = jnp.zeros((PK,), jnp.bfloat16)
    for k in range(DP):
      u_k = s_pk[k] + g_pk * (p_pk[k] - s_pk[k])
      t = t + _pack_row(q_v, e, k) * u_k
    inter = jnp.sum(_unpack_sum(t))

    return jnp.where(lane == (e % NUM_LANES), inter, out_chunk)

  def outer(j, out_chunk):
    for b in range(2):
      e = 2 * j + b
      wait_gather(e, b)
      out_chunk = compute(e, b, out_chunk)

      @pl.when(e + 2 < n_elems)
      def _():
        start_gather(e + 2, b)

      @pl.when(e % NUM_LANES == NUM_LANES - 1)
      def _():
        g0 = (e // NUM_LANES) * NUM_LANES
        out_v[pl.ds(g0, NUM_LANES)] = (
            out_chunk + bs_v[pl.ds(g0, NUM_LANES)]
            + bp_v[pl.ds(g0, NUM_LANES)] + GLOBAL_MEAN)
    return out_chunk

  lax.fori_loop(0, n_elems // 2, outer,
                jnp.zeros((NUM_LANES,), jnp.float32), unroll=False)

  pltpu.sync_copy(out_v, out_hbm.at[pl.ds(base, n_elems)])


def kernel(P, Q, Y, Bs, Bp, gate_W, gate_b, SIDs, PIDs, implicit_PIDs,
           implicit_lengths):
  B = SIDs.shape[0]
  n_workers = 32
  n_elems = B // n_workers

  sids = SIDs.astype(jnp.int32)
  pids = PIDs.astype(jnp.int32)
  lens = implicit_lengths.astype(jnp.int32)
  ipids = implicit_PIDs.astype(jnp.int32)
  gw = gate_W.reshape(2, D)
  gb = jnp.broadcast_to(gate_b.reshape(1), (NUM_LANES,)).astype(jnp.float32)

  mesh = plsc.VectorSubcoreMesh(core_axis_name="c", subcore_axis_name="s")
  cparams = pltpu.CompilerParams(needs_layout_passes=False,
                                 use_tc_tiling_on_sc=False)

  pack = pl.kernel(
      _pack_body,
      out_type=jax.ShapeDtypeStruct((Y.shape[0], D // 2), jnp.int32),
      mesh=mesh,
      compiler_params=cparams,
      scratch_types=[
          pltpu.VMEM((2, A_CH, D), jnp.float32),
          pltpu.VMEM((2, A_CH, D // 2), jnp.int32),
          pltpu.SemaphoreType.DMA,
          pltpu.SemaphoreType.DMA,
          pltpu.SemaphoreType.DMA,
          pltpu.SemaphoreType.DMA,
      ],
  )
  ypk = pack(Y)

  f = pl.kernel(
      functools.partial(_body, n_elems),
      out_type=jax.ShapeDtypeStruct((B,), jnp.float32),
      mesh=mesh,
      compiler_params=cparams,
      scratch_types=[
          pltpu.VMEM((n_elems,), jnp.int32),
          pltpu.VMEM((n_elems,), jnp.int32),
          pltpu.VMEM((n_elems + NUM_LANES,), jnp.int32),
          pltpu.VMEM((n_elems, 2, L_SPLIT), jnp.int32),
          pltpu.VMEM((n_elems, D), jnp.float32),
          pltpu.VMEM((n_elems, D), jnp.float32),
          pltpu.VMEM((n_elems,), jnp.float32),
          pltpu.VMEM((n_elems,), jnp.float32),
          pltpu.VMEM((2, D), jnp.float32),
          pltpu.VMEM((NUM_LANES,), jnp.float32),
          pltpu.VMEM((2, L_PAD, D // 2), jnp.int32),
          pltpu.VMEM((L_PAD,), jnp.float32),
          pltpu.VMEM((n_elems,), jnp.float32),
          pltpu.SemaphoreType.DMA,
          pltpu.SemaphoreType.DMA,
          pltpu.SemaphoreType.DMA,
      ],
  )
  return f(P, Q, ypk, Bs.reshape(-1), Bp.reshape(-1), gw, gb, sids, pids,
           ipids, lens)
